# Initial kernel scaffold; baseline (speedup 1.0000x reference)
import jax
import jax.numpy as jnp
from jax.experimental import pallas as pl


def kernel(node_features, edge_index, rank_mapping, W1, b1, W2, b2, Wf, bf):
    raise NotImplementedError("write your pallas kernel here")


# speedup vs baseline: 13.9815x; 13.9815x over previous
import functools
import jax
import jax.numpy as jnp
from jax import lax
from jax.experimental import pallas as pl
from jax.experimental.pallas import tpu as pltpu
from jax.experimental.pallas import tpu_sc as plsc

_N = 10000
_E = 320000
_D = 128
_NC = 2
_NS = 16
_NW = _NC * _NS
_GRP = 128
_G = -(-_E // (_NW * _GRP))
_E_PAD = _NW * _G * _GRP
_N_PAD = 10240
_RPT = _N_PAD // _NS
_ZROWS = 64


def _edge_pass(table, src_idx, dst_idx):
    mesh = plsc.VectorSubcoreMesh(core_axis_name="c", subcore_axis_name="s")

    @functools.partial(
        pl.kernel,
        mesh=mesh,
        out_type=jax.ShapeDtypeStruct((_NC, _N_PAD, _D), jnp.float32),
        scratch_types=[
            pltpu.VMEM((_G, _GRP), jnp.int32),
            pltpu.VMEM((_G, _GRP), jnp.int32),
            pltpu.VMEM((_GRP, _D), jnp.float32),
            pltpu.VMEM((_ZROWS, _D), jnp.float32),
            pltpu.VMEM_SHARED((_N_PAD, _D), jnp.float32),
            pltpu.SemaphoreType.DMA,
        ],
    )
    def k(table_hbm, src_hbm, dst_hbm, out_hbm, sidx, didx, rows, zbuf, acc, sem):
        c = lax.axis_index("c")
        s = lax.axis_index("s")
        wid = c * _NS + s
        zeros16 = jnp.zeros((16,), jnp.float32)

        def zrow(i, carry):
            for j in range(_D // 16):
                zbuf[i, pl.ds(j * 16, 16)] = zeros16
            return carry

        lax.fori_loop(0, _ZROWS, zrow, 0)

        base = s * _RPT
        for bb in range(_RPT // _ZROWS):
            pltpu.sync_copy(zbuf, acc.at[pl.ds(base + bb * _ZROWS, _ZROWS)])
        plsc.subcore_barrier()

        pltpu.sync_copy(src_hbm.at[wid], sidx)
        pltpu.sync_copy(dst_hbm.at[wid], didx)

        def body(g, carry):
            pltpu.async_copy(table_hbm.at[didx.at[g]], rows, sem).wait()
            pltpu.sync_copy(rows, acc.at[sidx.at[g]], add=True)
            return carry

        lax.fori_loop(0, _G, body, 0)

        plsc.subcore_barrier()
        pltpu.sync_copy(acc.at[pl.ds(base, _RPT)],
                        out_hbm.at[c, pl.ds(base, _RPT)])

    return k(table, src_idx, dst_idx)


def _mm_relu_k(x_ref, w_ref, b_ref, o_ref):
    y = jnp.dot(x_ref[...], w_ref[...], preferred_element_type=jnp.float32)
    o_ref[...] = jnp.maximum(y + b_ref[...], 0.0)


def _merge_mm_k(a_ref, w_ref, b_ref, o_ref):
    x = a_ref[0] + a_ref[1]
    y = jnp.dot(x, w_ref[...], preferred_element_type=jnp.float32)
    o_ref[...] = y + b_ref[...]


def _linear_relu(x, w, b):
    n = x.shape[0]
    blk = 1000
    return pl.pallas_call(
        _mm_relu_k,
        grid=(n // blk,),
        in_specs=[
            pl.BlockSpec((blk, _D), lambda i: (i, 0)),
            pl.BlockSpec((_D, _D), lambda i: (0, 0)),
            pl.BlockSpec((1, _D), lambda i: (0, 0)),
        ],
        out_specs=pl.BlockSpec((blk, _D), lambda i: (i, 0)),
        out_shape=jax.ShapeDtypeStruct((n, _D), jnp.float32),
    )(x, w, b)


def _merge_linear(acc, w, b):
    n = acc.shape[1]
    blk = 1280
    return pl.pallas_call(
        _merge_mm_k,
        grid=(n // blk,),
        in_specs=[
            pl.BlockSpec((_NC, blk, _D), lambda i: (0, i, 0)),
            pl.BlockSpec((_D, _D), lambda i: (0, 0)),
            pl.BlockSpec((1, _D), lambda i: (0, 0)),
        ],
        out_specs=pl.BlockSpec((blk, _D), lambda i: (i, 0)),
        out_shape=jax.ShapeDtypeStruct((n, _D), jnp.float32),
    )(acc, w, b)


def kernel(node_features, edge_index, rank_mapping, W1, b1, W2, b2, Wf, bf):
    del rank_mapping
    f = node_features[0].astype(jnp.float32)
    src = edge_index[0, 0, :].astype(jnp.int32)
    dst = edge_index[0, 1, :].astype(jnp.int32)
    pad = _E_PAD - _E
    src_p = jnp.concatenate(
        [src, jnp.full((pad,), _N, jnp.int32)]).reshape(_NW, _G, _GRP)
    dst_p = jnp.concatenate(
        [dst, jnp.zeros((pad,), jnp.int32)]).reshape(_NW, _G, _GRP)

    h1 = _linear_relu(f, W1.T, b1.reshape(1, _D))
    acc1 = _edge_pass(h1, src_p, dst_p)
    h2 = _merge_linear(acc1, W2.T, b2.reshape(1, _D))
    acc2 = _edge_pass(h2, src_p, dst_p)

    n_cls = Wf.shape[0]
    wf_pad = jnp.zeros((_D, _D), jnp.float32).at[:, :n_cls].set(Wf.T)
    bf_pad = jnp.zeros((1, _D), jnp.float32).at[0, :n_cls].set(bf)
    out = _merge_linear(acc2, wf_pad, bf_pad)
    return out[:_N, :n_cls][None]
